# Initial kernel scaffold; baseline (speedup 1.0000x reference)
import jax
import jax.numpy as jnp
from jax.experimental import pallas as pl


def kernel(x, pos_embed):
    raise NotImplementedError("write your pallas kernel here")


# speedup vs baseline: 2.8264x; 2.8264x over previous
import jax
import jax.numpy as jnp
from jax.experimental import pallas as pl


def _add_body(x_ref, p_ref, o_ref):
    o_ref[...] = x_ref[...] + p_ref[...]


def kernel(x, pos_embed):
    B, S, D = x.shape
    BLK = 512
    grid = (S // BLK, B)
    return pl.pallas_call(
        _add_body,
        grid=grid,
        in_specs=[
            pl.BlockSpec((1, BLK, D), lambda s, b: (b, s, 0)),
            pl.BlockSpec((BLK, D), lambda s, b: (s, 0)),
        ],
        out_specs=pl.BlockSpec((1, BLK, D), lambda s, b: (b, s, 0)),
        out_shape=jax.ShapeDtypeStruct(x.shape, x.dtype),
    )(x, pos_embed)


# speedup vs baseline: 3.1687x; 1.1211x over previous
import jax
import jax.numpy as jnp
from jax.experimental import pallas as pl


def _add_body(x_ref, p_ref, o_ref):
    o_ref[...] = x_ref[...] + p_ref[...]


def kernel(x, pos_embed):
    B, S, D = x.shape
    BLK = 1024
    grid = (S // BLK, B)
    return pl.pallas_call(
        _add_body,
        grid=grid,
        in_specs=[
            pl.BlockSpec((1, BLK, D), lambda s, b: (b, s, 0)),
            pl.BlockSpec((BLK, D), lambda s, b: (s, 0)),
        ],
        out_specs=pl.BlockSpec((1, BLK, D), lambda s, b: (b, s, 0)),
        out_shape=jax.ShapeDtypeStruct(x.shape, x.dtype),
    )(x, pos_embed)


# speedup vs baseline: 3.3063x; 1.0434x over previous
import jax
import jax.numpy as jnp
from jax.experimental import pallas as pl


def _add_body(x_ref, p_ref, o_ref):
    o_ref[...] = x_ref[...] + p_ref[...]


def kernel(x, pos_embed):
    B, S, D = x.shape
    BLK = 2048
    grid = (S // BLK, B)
    return pl.pallas_call(
        _add_body,
        grid=grid,
        in_specs=[
            pl.BlockSpec((1, BLK, D), lambda s, b: (b, s, 0)),
            pl.BlockSpec((BLK, D), lambda s, b: (s, 0)),
        ],
        out_specs=pl.BlockSpec((1, BLK, D), lambda s, b: (b, s, 0)),
        out_shape=jax.ShapeDtypeStruct(x.shape, x.dtype),
    )(x, pos_embed)


# speedup vs baseline: 3.3065x; 1.0001x over previous
import jax
import jax.numpy as jnp
from jax.experimental import pallas as pl


def _add_body(x_ref, p_ref, o_ref):
    o_ref[...] = x_ref[...] + p_ref[...]


def kernel(x, pos_embed):
    B, S, D = x.shape
    BLK = 2048
    n_s = S // BLK
    x2 = x.reshape(B * S, D)
    out = pl.pallas_call(
        _add_body,
        grid=(n_s, B),
        in_specs=[
            pl.BlockSpec((BLK, D), lambda s, b: (b * n_s + s, 0)),
            pl.BlockSpec((BLK, D), lambda s, b: (s, 0)),
        ],
        out_specs=pl.BlockSpec((BLK, D), lambda s, b: (b * n_s + s, 0)),
        out_shape=jax.ShapeDtypeStruct((B * S, D), x.dtype),
    )(x2, pos_embed)
    return out.reshape(B, S, D)
